# Initial kernel scaffold; baseline (speedup 1.0000x reference)
import jax
import jax.numpy as jnp
from jax.experimental import pallas as pl


def kernel(points, features, random_shift):
    raise NotImplementedError("write your pallas kernel here")


# speedup vs baseline: 2.5978x; 2.5978x over previous
import functools

import numpy as np
import jax
import jax.numpy as jnp
from jax import lax
from jax.experimental import pallas as pl
from jax.experimental.pallas import tpu as pltpu
from jax.experimental.pallas import tpu_sc as plsc

_POS_DIM = 3
_NR_LEVELS = 16
_NR_FEAT = 2
_LOG2_HASH = 19
_CAPACITY = 2 ** _LOG2_HASH
_N_POINTS = 262144
_SCALES = np.geomspace(1.0, 1e-4, num=_NR_LEVELS).astype(np.float32)

_NC, _NS = 2, 16
_NW = _NC * _NS
_NPT = _N_POINTS // _NW
_C = 1024
_NCHUNK = _NPT // _C
_NG = _C // 16

_P1 = np.int32(np.uint32(2654435761))
_P2 = np.int32(805459861)


def _f(v):
    return jnp.full((16,), v, jnp.float32)


def _bc(s):
    return jnp.broadcast_to(s, (16,))


def _i(v):
    return jnp.full((16,), v, jnp.int32)


def _sc_body(px, py, pz, feats, af_hbm, bf_hbm, out_hbm,
             x_v, y_v, z_v, af_v, bf_v, idx_v, bary_v, rows_v, out_v, sem):
    cid = lax.axis_index("c")
    sid = lax.axis_index("s")
    wid = sid * _NC + cid

    pltpu.sync_copy(af_hbm, af_v)
    pltpu.sync_copy(bf_hbm, bf_v)

    ZERO, ONE, QUARTER, FOUR = _f(0.0), _f(1.0), _f(0.25), _f(4.0)
    IZERO, IONE, ITWO = _i(0), _i(1), _i(2)
    I3, I4, IM4 = _i(3), _i(4), _i(-4)
    MASK = _i(_CAPACITY - 1)
    P1V, P2V = _i(_P1), _i(_P2)
    lane = lax.iota(jnp.int32, 16)
    lane32 = lane * _i(32)

    def chunk_body(ci, carry):
        base = wid * _NPT + ci * _C
        pltpu.sync_copy(px.at[pl.ds(base, _C)], x_v)
        pltpu.sync_copy(py.at[pl.ds(base, _C)], y_v)
        pltpu.sync_copy(pz.at[pl.ds(base, _C)], z_v)
        def level_body(l, carry2):
            lvec = _bc(l)
            a0 = plsc.load_gather(af_v, [lvec])
            a1 = plsc.load_gather(af_v, [lvec + _i(16)])
            a2 = plsc.load_gather(af_v, [lvec + _i(32)])
            b0 = plsc.load_gather(bf_v, [lvec])
            b1 = plsc.load_gather(bf_v, [lvec + _i(16)])
            b2 = plsc.load_gather(bf_v, [lvec + _i(32)])
            loff = lvec * _i(_CAPACITY)

            def grp_body(i, carry3):
                o = i * 16
                x = x_v[pl.ds(o, 16)]
                y = y_v[pl.ds(o, 16)]
                z = z_v[pl.ds(o, 16)]
                cf0 = x * a0 + b0
                cf1 = y * a1 + b1
                cf2 = z * a2 + b2
                e = [cf0 + cf1 + cf2,
                     cf1 + cf2 - cf0,
                     cf2 - (cf1 + cf1),
                     -(cf2 + cf2 + cf2)]
                rem0 = []
                for j in range(4):
                    v = e[j] * QUARTER
                    tf = v.astype(jnp.int32).astype(jnp.float32)
                    fl = tf - jnp.where(tf > v, ONE, ZERO)
                    up = (fl + jnp.where(v != fl, ONE, ZERO)) * FOUR
                    down = fl * FOUR
                    rem0.append(jnp.where((up - e[j]) < (e[j] - down), up, down))
                sum_i = ((rem0[0] + rem0[1] + rem0[2] + rem0[3])
                         * QUARTER).astype(jnp.int32)
                d0 = [e[j] - rem0[j] for j in range(4)]
                rank = [sum_i, sum_i, sum_i, sum_i]
                for a in range(4):
                    for b in range(a + 1, 4):
                        less = d0[a] < d0[b]
                        rank[a] = rank[a] + jnp.where(less, IONE, IZERO)
                        rank[b] = rank[b] + jnp.where(less, IZERO, IONE)
                rem0i = [rem0[j].astype(jnp.int32) for j in range(4)]
                for j in range(4):
                    adj = jnp.where(rank[j] < 0, I4,
                                    jnp.where(rank[j] > I3, IM4, IZERO))
                    rank[j] = rank[j] + adj
                    rem0i[j] = rem0i[j] + adj
                delta = [(e[j] - rem0i[j].astype(jnp.float32)) * QUARTER
                         for j in range(4)]
                s = []
                for c in range(4):
                    cc = _i(c)
                    acc = jnp.where(rank[0] == cc, delta[0], ZERO)
                    for j in range(1, 4):
                        acc = acc + jnp.where(rank[j] == cc, delta[j], ZERO)
                    s.append(acc)
                bary = [ONE + s[3] - s[0], s[2] - s[3], s[1] - s[2], s[0] - s[1]]
                for rem in range(4):
                    if rem == 0:
                        k0, k1, k2 = rem0i[0], rem0i[1], rem0i[2]
                    else:
                        thr = _i(3 - rem)
                        radd, rsub = _i(rem), _i(rem - 4)
                        k0 = rem0i[0] + jnp.where(rank[0] > thr, rsub, radd)
                        k1 = rem0i[1] + jnp.where(rank[1] > thr, rsub, radd)
                        k2 = rem0i[2] + jnp.where(rank[2] > thr, rsub, radd)
                    h = (k0 ^ (k1 * P1V) ^ (k2 * P2V)) & MASK
                    gidx = (h + loff) * ITWO
                    idx_v[pl.ds((2 * rem) * _C + o, 16)] = gidx
                    idx_v[pl.ds((2 * rem + 1) * _C + o, 16)] = gidx + IONE
                    bary_v[pl.ds(rem * _C + o, 16)] = bary[rem]
                return carry3

            lax.fori_loop(0, _NG, grp_body, 0, unroll=False)
            pltpu.async_copy(feats.at[idx_v], rows_v, sem).wait()

            def acc_body(i, carry3):
                o = i * 16
                out0 = ZERO
                out1 = ZERO
                for rem in range(4):
                    w = bary_v[pl.ds(rem * _C + o, 16)]
                    f0 = rows_v[pl.ds((2 * rem) * _C + o, 16)]
                    f1 = rows_v[pl.ds((2 * rem + 1) * _C + o, 16)]
                    out0 = out0 + w * f0
                    out1 = out1 + w * f1
                pos = lane32 + _bc(o * 32 + 2 * l)
                plsc.store_scatter(out_v, [pos], out0)
                plsc.store_scatter(out_v, [pos + IONE], out1)
                return carry3

            lax.fori_loop(0, _NG, acc_body, 0, unroll=False)
            return carry2

        lax.fori_loop(0, _NR_LEVELS, level_body, 0, unroll=False)
        pltpu.sync_copy(out_v, out_hbm.at[pl.ds(base * 32, _C * 32)])
        return carry

    lax.fori_loop(0, _NCHUNK, chunk_body, 0, unroll=False)


@jax.jit
def _encode(px, py, pz, feats, af, bf):
    mesh = plsc.VectorSubcoreMesh(core_axis_name="c", subcore_axis_name="s")
    fn = functools.partial(
        pl.kernel, mesh=mesh,
        compiler_params=pltpu.CompilerParams(
            needs_layout_passes=False, use_tc_tiling_on_sc=False),
        out_type=jax.ShapeDtypeStruct((_N_POINTS * 32,), jnp.float32),
        scratch_types=[
            pltpu.VMEM((_C,), jnp.float32),
            pltpu.VMEM((_C,), jnp.float32),
            pltpu.VMEM((_C,), jnp.float32),
            pltpu.VMEM((_POS_DIM * _NR_LEVELS,), jnp.float32),
            pltpu.VMEM((_POS_DIM * _NR_LEVELS,), jnp.float32),
            pltpu.VMEM((8 * _C,), jnp.int32),
            pltpu.VMEM((4 * _C,), jnp.float32),
            pltpu.VMEM((8 * _C,), jnp.float32),
            pltpu.VMEM((32 * _C,), jnp.float32),
            pltpu.SemaphoreType.DMA,
        ],
    )(_sc_body)
    return fn(px, py, pz, feats, af, bf)


def kernel(points, features, random_shift):
    sf = (1.0 / np.sqrt((np.arange(_POS_DIM) + 1.0)
                        * (np.arange(_POS_DIM) + 2.0))).astype(np.float32)
    af = jnp.asarray((sf[None, :] / _SCALES[:, None]).T.reshape(-1),
                     dtype=jnp.float32)
    bf = (random_shift * sf[None, :]).astype(jnp.float32).T.reshape(-1)
    px = points[:, 0]
    py = points[:, 1]
    pz = points[:, 2]
    feats = features.reshape(_NR_LEVELS * _CAPACITY * _NR_FEAT)
    out = _encode(px, py, pz, feats, af, bf)
    return out.reshape(_N_POINTS, _NR_LEVELS * _NR_FEAT)


# speedup vs baseline: 11.8034x; 4.5437x over previous
import functools

import numpy as np
import jax
import jax.numpy as jnp
from jax import lax
from jax.experimental import pallas as pl
from jax.experimental.pallas import tpu as pltpu
from jax.experimental.pallas import tpu_sc as plsc

_POS_DIM = 3
_NR_LEVELS = 16
_NR_FEAT = 2
_LOG2_HASH = 19
_CAPACITY = 2 ** _LOG2_HASH
_N_POINTS = 262144
_SCALES = np.geomspace(1.0, 1e-4, num=_NR_LEVELS).astype(np.float32)

_NC, _NS = 2, 16
_NW = _NC * _NS
_NPT = _N_POINTS // _NW
_C = 1024
_NCHUNK = _NPT // _C
_NG = _C // 16

_P1 = np.int32(np.uint32(2654435761))
_P2 = np.int32(805459861)


def _f(v):
    return jnp.full((16,), v, jnp.float32)


def _bc(s):
    return jnp.broadcast_to(s, (16,))


def _i(v):
    return jnp.full((16,), v, jnp.int32)


def _sc_body(px, py, pz, feats, af_hbm, bf_hbm, out_hbm,
             x_v, y_v, z_v, af_v, bf_v, idx_v, bary_v, rows_v, out_v, sem):
    cid = lax.axis_index("c")
    sid = lax.axis_index("s")
    wid = sid * _NC + cid

    pltpu.sync_copy(af_hbm, af_v)
    pltpu.sync_copy(bf_hbm, bf_v)

    ZERO, ONE, QUARTER, FOUR = _f(0.0), _f(1.0), _f(0.25), _f(4.0)
    IZERO, IONE = _i(0), _i(1)
    FOFF = _i(_NR_LEVELS * _CAPACITY)
    I3, I4, IM4 = _i(3), _i(4), _i(-4)
    MASK = _i(_CAPACITY - 1)
    P1V, P2V = _i(_P1), _i(_P2)
    lane = lax.iota(jnp.int32, 16)
    lane32 = lane * _i(32)

    def chunk_body(ci, carry):
        base = wid * _NPT + ci * _C
        pltpu.sync_copy(px.at[pl.ds(base, _C)], x_v)
        pltpu.sync_copy(py.at[pl.ds(base, _C)], y_v)
        pltpu.sync_copy(pz.at[pl.ds(base, _C)], z_v)
        def level_body(l, carry2):
            lvec = _bc(l)
            a0 = plsc.load_gather(af_v, [lvec])
            a1 = plsc.load_gather(af_v, [lvec + _i(16)])
            a2 = plsc.load_gather(af_v, [lvec + _i(32)])
            b0 = plsc.load_gather(bf_v, [lvec])
            b1 = plsc.load_gather(bf_v, [lvec + _i(16)])
            b2 = plsc.load_gather(bf_v, [lvec + _i(32)])
            loff = lvec * _i(_CAPACITY)

            def grp_body(i, carry3):
                o = i * 16
                x = x_v[pl.ds(o, 16)]
                y = y_v[pl.ds(o, 16)]
                z = z_v[pl.ds(o, 16)]
                cf0 = x * a0 + b0
                cf1 = y * a1 + b1
                cf2 = z * a2 + b2
                e = [cf0 + cf1 + cf2,
                     cf1 + cf2 - cf0,
                     cf2 - (cf1 + cf1),
                     -(cf2 + cf2 + cf2)]
                rem0 = []
                for j in range(4):
                    v = e[j] * QUARTER
                    tf = v.astype(jnp.int32).astype(jnp.float32)
                    fl = tf - jnp.where(tf > v, ONE, ZERO)
                    up = (fl + jnp.where(v != fl, ONE, ZERO)) * FOUR
                    down = fl * FOUR
                    rem0.append(jnp.where((up - e[j]) < (e[j] - down), up, down))
                sum_i = ((rem0[0] + rem0[1] + rem0[2] + rem0[3])
                         * QUARTER).astype(jnp.int32)
                d0 = [e[j] - rem0[j] for j in range(4)]
                rank = [sum_i, sum_i, sum_i, sum_i]
                for a in range(4):
                    for b in range(a + 1, 4):
                        less = d0[a] < d0[b]
                        rank[a] = rank[a] + jnp.where(less, IONE, IZERO)
                        rank[b] = rank[b] + jnp.where(less, IZERO, IONE)
                rem0i = [rem0[j].astype(jnp.int32) for j in range(4)]
                for j in range(4):
                    adj = jnp.where(rank[j] < 0, I4,
                                    jnp.where(rank[j] > I3, IM4, IZERO))
                    rank[j] = rank[j] + adj
                    rem0i[j] = rem0i[j] + adj
                delta = [(e[j] - rem0i[j].astype(jnp.float32)) * QUARTER
                         for j in range(4)]
                s = []
                for c in range(4):
                    cc = _i(c)
                    acc = jnp.where(rank[0] == cc, delta[0], ZERO)
                    for j in range(1, 4):
                        acc = acc + jnp.where(rank[j] == cc, delta[j], ZERO)
                    s.append(acc)
                bary = [ONE + s[3] - s[0], s[2] - s[3], s[1] - s[2], s[0] - s[1]]
                for rem in range(4):
                    if rem == 0:
                        k0, k1, k2 = rem0i[0], rem0i[1], rem0i[2]
                    else:
                        thr = _i(3 - rem)
                        radd, rsub = _i(rem), _i(rem - 4)
                        k0 = rem0i[0] + jnp.where(rank[0] > thr, rsub, radd)
                        k1 = rem0i[1] + jnp.where(rank[1] > thr, rsub, radd)
                        k2 = rem0i[2] + jnp.where(rank[2] > thr, rsub, radd)
                    h = (k0 ^ (k1 * P1V) ^ (k2 * P2V)) & MASK
                    gidx = h + loff
                    idx_v[pl.ds((2 * rem) * _C + o, 16)] = gidx
                    idx_v[pl.ds((2 * rem + 1) * _C + o, 16)] = gidx + FOFF
                    bary_v[pl.ds(rem * _C + o, 16)] = bary[rem]
                return carry3

            lax.fori_loop(0, _NG, grp_body, 0, unroll=False)
            pltpu.async_copy(feats.at[idx_v], rows_v, sem).wait()

            def acc_body(i, carry3):
                o = i * 16
                out0 = ZERO
                out1 = ZERO
                for rem in range(4):
                    w = bary_v[pl.ds(rem * _C + o, 16)]
                    f0 = rows_v[pl.ds((2 * rem) * _C + o, 16)]
                    f1 = rows_v[pl.ds((2 * rem + 1) * _C + o, 16)]
                    out0 = out0 + w * f0
                    out1 = out1 + w * f1
                pos = lane32 + _bc(o * 32 + 2 * l)
                plsc.store_scatter(out_v, [pos], out0)
                plsc.store_scatter(out_v, [pos + IONE], out1)
                return carry3

            lax.fori_loop(0, _NG, acc_body, 0, unroll=False)
            return carry2

        lax.fori_loop(0, _NR_LEVELS, level_body, 0, unroll=False)
        pltpu.sync_copy(out_v, out_hbm.at[pl.ds(base * 32, _C * 32)])
        return carry

    lax.fori_loop(0, _NCHUNK, chunk_body, 0, unroll=False)


@jax.jit
def _encode(px, py, pz, feats, af, bf):
    mesh = plsc.VectorSubcoreMesh(core_axis_name="c", subcore_axis_name="s")
    fn = functools.partial(
        pl.kernel, mesh=mesh,
        compiler_params=pltpu.CompilerParams(
            needs_layout_passes=False, use_tc_tiling_on_sc=False),
        out_type=jax.ShapeDtypeStruct((_N_POINTS * 32,), jnp.float32),
        scratch_types=[
            pltpu.VMEM((_C,), jnp.float32),
            pltpu.VMEM((_C,), jnp.float32),
            pltpu.VMEM((_C,), jnp.float32),
            pltpu.VMEM((_POS_DIM * _NR_LEVELS,), jnp.float32),
            pltpu.VMEM((_POS_DIM * _NR_LEVELS,), jnp.float32),
            pltpu.VMEM((8 * _C,), jnp.int32),
            pltpu.VMEM((4 * _C,), jnp.float32),
            pltpu.VMEM((8 * _C,), jnp.float32),
            pltpu.VMEM((32 * _C,), jnp.float32),
            pltpu.SemaphoreType.DMA,
        ],
    )(_sc_body)
    return fn(px, py, pz, feats, af, bf)


def kernel(points, features, random_shift):
    sf = (1.0 / np.sqrt((np.arange(_POS_DIM) + 1.0)
                        * (np.arange(_POS_DIM) + 2.0))).astype(np.float32)
    af = jnp.asarray((sf[None, :] / _SCALES[:, None]).T.reshape(-1),
                     dtype=jnp.float32)
    bf = (random_shift * sf[None, :]).astype(jnp.float32).T.reshape(-1)
    px = points[:, 0]
    py = points[:, 1]
    pz = points[:, 2]
    feats = jnp.concatenate([features[:, :, 0].reshape(-1),
                             features[:, :, 1].reshape(-1)])
    out = _encode(px, py, pz, feats, af, bf)
    return out.reshape(_N_POINTS, _NR_LEVELS * _NR_FEAT)
